# Initial kernel scaffold; baseline (speedup 1.0000x reference)
import jax
import jax.numpy as jnp
from jax.experimental import pallas as pl


def kernel(x, conv_w, conv_b, gamma, beta):
    raise NotImplementedError("write your pallas kernel here")


# speedup vs baseline: 2.1164x; 2.1164x over previous
import functools

import jax
import jax.numpy as jnp
from jax.experimental import pallas as pl
from jax.experimental.pallas import tpu as pltpu


def _conv_stats_kernel(H, W, x_ref, w_ref, b_ref, y_ref, s_ref, ss_ref,
                       xs_ref):
    HW = H * W
    cin = x_ref.shape[1]
    pad = xs_ref.shape[-1]

    xb = x_ref[0].astype(jnp.bfloat16)
    for ky in range(3):
        off = (2 - ky) * W
        r0 = ky * cin
        if off > 0:
            xs_ref[r0:r0 + cin, :off] = jnp.zeros((cin, off), jnp.bfloat16)
        xs_ref[r0:r0 + cin, off:off + HW] = xb
        tail = pad - off - HW
        if tail > 0:
            xs_ref[r0:r0 + cin, off + HW:] = jnp.zeros((cin, tail),
                                                       jnp.bfloat16)

    col = jax.lax.broadcasted_iota(jnp.int32, (1, HW), 1) % W

    acc = jnp.zeros((w_ref.shape[1], HW), jnp.float32)
    for kx in range(3):
        sl = xs_ref[:, W - 1 + kx:W - 1 + kx + HW]
        if kx == 0:
            sl = jnp.where(col == 0, jnp.bfloat16(0), sl)
        elif kx == 2:
            sl = jnp.where(col == W - 1, jnp.bfloat16(0), sl)
        acc = acc + jnp.dot(w_ref[kx], sl,
                            preferred_element_type=jnp.float32)

    acc = jnp.maximum(acc + b_ref[...], 0.0)
    y_ref[0] = acc.astype(jnp.bfloat16)
    s_ref[0] = jnp.sum(acc, axis=1, keepdims=True)
    ss_ref[0] = jnp.sum(acc * acc, axis=1, keepdims=True)


def _bn_apply_kernel(y_ref, sc_ref, sh_ref, o_ref):
    o_ref[0] = y_ref[0].astype(jnp.float32) * sc_ref[...] + sh_ref[...]


def kernel(x, conv_w, conv_b, gamma, beta, eps=1e-5):
    N, Cin, H, Wd = x.shape
    Cout = conv_w.shape[0]
    HW = H * Wd
    pad = -(-(HW + 2 * Wd) // 128) * 128

    xf = x.reshape(N, Cin, HW)
    w3 = jnp.transpose(conv_w, (3, 0, 2, 1)).reshape(3, Cout, 3 * Cin)
    w3 = w3.astype(jnp.bfloat16)
    b2 = conv_b.reshape(Cout, 1).astype(jnp.float32)

    cparams = pltpu.CompilerParams(
        dimension_semantics=("parallel",),
        vmem_limit_bytes=64 * 1024 * 1024)

    y, s, ss = pl.pallas_call(
        functools.partial(_conv_stats_kernel, H, Wd),
        out_shape=(jax.ShapeDtypeStruct((N, Cout, HW), jnp.bfloat16),
                   jax.ShapeDtypeStruct((N, Cout, 1), jnp.float32),
                   jax.ShapeDtypeStruct((N, Cout, 1), jnp.float32)),
        grid_spec=pltpu.PrefetchScalarGridSpec(
            num_scalar_prefetch=0,
            grid=(N,),
            in_specs=[
                pl.BlockSpec((1, Cin, HW), lambda n: (n, 0, 0)),
                pl.BlockSpec((3, Cout, 3 * Cin), lambda n: (0, 0, 0)),
                pl.BlockSpec((Cout, 1), lambda n: (0, 0)),
            ],
            out_specs=[
                pl.BlockSpec((1, Cout, HW), lambda n: (n, 0, 0)),
                pl.BlockSpec((1, Cout, 1), lambda n: (n, 0, 0)),
                pl.BlockSpec((1, Cout, 1), lambda n: (n, 0, 0)),
            ],
            scratch_shapes=[pltpu.VMEM((3 * Cin, pad), jnp.bfloat16)]),
        compiler_params=cparams,
    )(xf, w3, b2)

    count = N * HW
    mean = jnp.sum(s[:, :, 0], axis=0) / count
    var = jnp.sum(ss[:, :, 0], axis=0) / count - mean * mean
    scale = gamma / jnp.sqrt(var + eps)
    shift = beta - mean * scale

    out = pl.pallas_call(
        _bn_apply_kernel,
        out_shape=jax.ShapeDtypeStruct((N, Cout, HW), jnp.float32),
        grid_spec=pltpu.PrefetchScalarGridSpec(
            num_scalar_prefetch=0,
            grid=(N,),
            in_specs=[
                pl.BlockSpec((1, Cout, HW), lambda n: (n, 0, 0)),
                pl.BlockSpec((Cout, 1), lambda n: (0, 0)),
                pl.BlockSpec((Cout, 1), lambda n: (0, 0)),
            ],
            out_specs=pl.BlockSpec((1, Cout, HW), lambda n: (n, 0, 0))),
        compiler_params=cparams,
    )(y, scale.reshape(Cout, 1), shift.reshape(Cout, 1))

    return out.reshape(N, Cout, H, Wd)
